# baseline (speedup 1.0000x reference)
import functools

import jax
import jax.numpy as jnp
from jax import lax
from jax.experimental import pallas as pl
from jax.experimental.pallas import tpu as pltpu
from jax.experimental.pallas import tpu_sc as plsc

EPS = 1e-5
L = 16
NC, NS = 2, 16
NW = NC * NS
TOPK = 8


def _score_body(x_ref, m_ref, wbf_ref, gt_ref, bet_ref, bc_ref, s_ref, *, D):
    xbt = x_ref[0].T
    mu = jnp.sum(xbt, axis=0, keepdims=True) * (1.0 / D)
    xc = xbt - mu
    var = jnp.sum(xc * xc, axis=0, keepdims=True) * (1.0 / D)
    xnt = xc / jnp.sqrt(var + EPS) * gt_ref[...] + bet_ref[...]
    xnb = xnt.astype(jnp.bfloat16)
    lgt = jnp.dot(wbf_ref[...], xnb,
                  preferred_element_type=jnp.float32)
    lgt = lgt + bc_ref[...]
    sc = jnp.max(lgt, axis=0)
    sc = jnp.where(m_ref[...] != 0, sc, -jnp.inf)
    s_ref[...] = sc


def _scores(x, mask_f, wbf, gt, bet, bc, b0, bg):
    B, N, D = x.shape
    C = wbf.shape[0]
    return pl.pallas_call(
        functools.partial(_score_body, D=D),
        grid=(bg,),
        in_specs=[
            pl.BlockSpec((1, N, D), lambda i: (b0 + i, 0, 0)),
            pl.BlockSpec((N,), lambda i: (b0 + i,)),
            pl.BlockSpec((C, D), lambda i: (0, 0)),
            pl.BlockSpec((D, 1), lambda i: (0, 0)),
            pl.BlockSpec((D, 1), lambda i: (0, 0)),
            pl.BlockSpec((C, 1), lambda i: (0, 0)),
        ],
        out_specs=pl.BlockSpec((N,), lambda i: (i,)),
        out_shape=jax.ShapeDtypeStruct((bg * N,), jnp.float32),
    )(x, mask_f, wbf, gt, bet, bc)


def _sc_select(score_flat, x_flat, b0, bg, N, D):
    bags_per = bg // NW
    n_chunks = N // L
    mesh = plsc.VectorSubcoreMesh(core_axis_name="c", subcore_axis_name="s")

    @functools.partial(
        pl.kernel,
        mesh=mesh,
        out_type=jax.ShapeDtypeStruct((bg * L, D), jnp.float32),
        scratch_types=[
            pltpu.VMEM((N,), jnp.float32),
            pltpu.VMEM((L,), jnp.int32),
            pltpu.VMEM((L, D), jnp.float32),
            pltpu.SemaphoreType.DMA,
        ],
        compiler_params=pltpu.CompilerParams(needs_layout_passes=False),
    )
    def k(score_hbm, x_hbm, out_hbm, sv, idxv, rows, sem):
        wid = lax.axis_index("s") * NC + lax.axis_index("c")
        for j in range(bags_per):
            b = wid * bags_per + j
            pltpu.sync_copy(score_hbm.at[pl.ds(b * N, N)], sv)

            neg = jnp.full((L,), -jnp.inf, jnp.float32)
            zero = jnp.zeros((L,), jnp.int32)
            init = (neg,) * TOPK + (zero,) * TOPK

            def body(i, carry):
                ts, cs = carry[:TOPK], carry[TOPK:]
                cur = sv[pl.ds(i * L, L)]
                curi = jnp.full((L,), i, jnp.int32)
                nts, ncs = [], []
                for t, c in zip(ts, cs):
                    m = cur > t
                    hi = jnp.maximum(t, cur)
                    lo = jnp.minimum(t, cur)
                    nts.append(hi)
                    ncs.append(jnp.where(m, curi, c))
                    curi = jnp.where(m, c, curi)
                    cur = lo
                return tuple(nts) + tuple(ncs)

            carry = lax.fori_loop(0, n_chunks, body, init)
            ts, cs = carry[:TOPK], carry[TOPK:]

            lane = lax.iota(jnp.int32, L)
            pairs = [plsc.sort_key_val(ts[q], cs[q] * L + lane)
                     for q in range(TOPK)]

            def merge(pa, pb):
                ka, va = pa
                kb, vb = pb
                rk = lax.rev(kb, (0,))
                rv = lax.rev(vb, (0,))
                m = ka >= rk
                hk = jnp.maximum(ka, rk)
                hv = jnp.where(m, va, rv)
                return plsc.sort_key_val(hk, hv)

            while len(pairs) > 1:
                pairs = [merge(pairs[q], pairs[q + 1])
                         for q in range(0, len(pairs), 2)]
            _, vidx = pairs[0]

            idxv[...] = vidx + (b0 + b) * N
            pltpu.async_copy(x_hbm.at[idxv], rows, sem).wait()
            pltpu.sync_copy(rows, out_hbm.at[pl.ds(b * L, L)])

    return k(score_flat, x_flat)


def _final_body(xs_ref, g_ref, be_ref, wtbf_ref, b_ref, o_ref):
    B = xs_ref.shape[0]
    C = wtbf_ref.shape[1]
    acc = jnp.zeros((B, C), jnp.float32)
    for j in range(L - TOPK, L):
        xr = xs_ref[:, j, :]
        mu = jnp.mean(xr, axis=1, keepdims=True)
        xc = xr - mu
        var = jnp.mean(xc * xc, axis=1, keepdims=True)
        xn = xc / jnp.sqrt(var + EPS) * g_ref[0] + be_ref[0]
        acc = acc + jnp.dot(xn.astype(jnp.bfloat16), wtbf_ref[...],
                            preferred_element_type=jnp.float32)
    o_ref[...] = acc * (1.0 / TOPK) + b_ref[0]


def _final(xsel, g2, be2, wt, b2):
    B = xsel.shape[0]
    D = xsel.shape[2]
    C = wt.shape[1]
    return pl.pallas_call(
        _final_body,
        in_specs=[
            pl.BlockSpec((B, L, D), lambda: (0, 0, 0)),
            pl.BlockSpec((1, D), lambda: (0, 0)),
            pl.BlockSpec((1, D), lambda: (0, 0)),
            pl.BlockSpec((D, C), lambda: (0, 0)),
            pl.BlockSpec((1, C), lambda: (0, 0)),
        ],
        out_specs=pl.BlockSpec((B, C), lambda: (0, 0)),
        out_shape=jax.ShapeDtypeStruct((B, C), jnp.float32),
    )(xsel, g2, be2, wt, b2)


def kernel(x, mask, ln_gamma, ln_beta, W, b):
    B, N, D = x.shape
    C = W.shape[0]
    mask_f = mask.reshape(B * N).astype(jnp.float32)
    g2 = ln_gamma.reshape(1, D)
    be2 = ln_beta.reshape(1, D)
    wbf = W.astype(jnp.bfloat16)
    wtbf = wbf.T
    gt = ln_gamma.reshape(D, 1)
    bet = ln_beta.reshape(D, 1)
    bc = b.reshape(C, 1)
    b2 = b.reshape(1, C)
    G = 2
    bg = B // G
    xf = x.reshape(B * N, D)
    xsels = []
    for g in range(G):
        sg = _scores(x, mask_f, wbf, gt, bet, bc, g * bg, bg)
        xsels.append(_sc_select(sg, xf, g * bg, bg, N, D))
    xsel = jnp.concatenate(xsels, axis=0)
    return _final(xsel.reshape(B, L, D), g2, be2, wtbf, b2)


# speedup vs baseline: 1.2554x; 1.2554x over previous
import functools

import jax
import jax.numpy as jnp
from jax import lax
from jax.experimental import pallas as pl
from jax.experimental.pallas import tpu as pltpu
from jax.experimental.pallas import tpu_sc as plsc

EPS = 1e-5
L = 16
NC, NS = 2, 16
NW = NC * NS
TOPK = 8


def _score_body(x_ref, m_ref, wbf_ref, gt_ref, bet_ref, bc_ref, s_ref, *, D):
    xbt = x_ref[0].T
    mu = jnp.sum(xbt, axis=0, keepdims=True) * (1.0 / D)
    xc = xbt - mu
    var = jnp.sum(xc * xc, axis=0, keepdims=True) * (1.0 / D)
    xnt = xc / jnp.sqrt(var + EPS) * gt_ref[...] + bet_ref[...]
    xnb = xnt.astype(jnp.bfloat16)
    lgt = jnp.dot(wbf_ref[...], xnb,
                  preferred_element_type=jnp.float32)
    lgt = lgt + bc_ref[...]
    sc = jnp.max(lgt, axis=0)
    sc = jnp.where(m_ref[...] != 0, sc, -jnp.inf)
    s_ref[...] = sc


def _scores(x, mask_f, wbf, gt, bet, bc, bpb=2):
    B, N, D = x.shape
    C = wbf.shape[0]
    nblk = B // bpb
    rows = bpb * N
    xr = x.reshape(nblk, rows, D)
    return pl.pallas_call(
        functools.partial(_score_body, D=D),
        grid=(nblk,),
        in_specs=[
            pl.BlockSpec((1, rows, D), lambda i: (i, 0, 0)),
            pl.BlockSpec((rows,), lambda i: (i,)),
            pl.BlockSpec((C, D), lambda i: (0, 0)),
            pl.BlockSpec((D, 1), lambda i: (0, 0)),
            pl.BlockSpec((D, 1), lambda i: (0, 0)),
            pl.BlockSpec((C, 1), lambda i: (0, 0)),
        ],
        out_specs=pl.BlockSpec((rows,), lambda i: (i,)),
        out_shape=jax.ShapeDtypeStruct((B * N,), jnp.float32),
    )(xr, mask_f, wbf, gt, bet, bc)


def _sc_select(score_flat, x_flat, b0, bg, N, D):
    bags_per = bg // NW
    n_chunks = N // L
    mesh = plsc.VectorSubcoreMesh(core_axis_name="c", subcore_axis_name="s")

    @functools.partial(
        pl.kernel,
        mesh=mesh,
        out_type=jax.ShapeDtypeStruct((bg * L, D), jnp.float32),
        scratch_types=[
            pltpu.VMEM((N,), jnp.float32),
            pltpu.VMEM((L,), jnp.int32),
            pltpu.VMEM((L, D), jnp.float32),
            pltpu.SemaphoreType.DMA,
        ],
        compiler_params=pltpu.CompilerParams(needs_layout_passes=False),
    )
    def k(score_hbm, x_hbm, out_hbm, sv, idxv, rows, sem):
        wid = lax.axis_index("s") * NC + lax.axis_index("c")
        for j in range(bags_per):
            b = wid * bags_per + j
            pltpu.sync_copy(score_hbm.at[pl.ds(b * N, N)], sv)

            neg = jnp.full((L,), -jnp.inf, jnp.float32)
            zero = jnp.zeros((L,), jnp.int32)
            init = (neg,) * TOPK + (zero,) * TOPK

            def body(i, carry):
                ts, cs = carry[:TOPK], carry[TOPK:]
                cur = sv[pl.ds(i * L, L)]
                curi = jnp.full((L,), i, jnp.int32)
                nts, ncs = [], []
                for t, c in zip(ts, cs):
                    m = cur > t
                    hi = jnp.maximum(t, cur)
                    lo = jnp.minimum(t, cur)
                    nts.append(hi)
                    ncs.append(jnp.where(m, curi, c))
                    curi = jnp.where(m, c, curi)
                    cur = lo
                return tuple(nts) + tuple(ncs)

            carry = lax.fori_loop(0, n_chunks, body, init)
            ts, cs = carry[:TOPK], carry[TOPK:]

            lane = lax.iota(jnp.int32, L)
            pairs = [plsc.sort_key_val(ts[q], cs[q] * L + lane)
                     for q in range(TOPK)]

            def merge(pa, pb):
                ka, va = pa
                kb, vb = pb
                rk = lax.rev(kb, (0,))
                rv = lax.rev(vb, (0,))
                m = ka >= rk
                hk = jnp.maximum(ka, rk)
                hv = jnp.where(m, va, rv)
                return plsc.sort_key_val(hk, hv)

            while len(pairs) > 1:
                pairs = [merge(pairs[q], pairs[q + 1])
                         for q in range(0, len(pairs), 2)]
            _, vidx = pairs[0]

            idxv[...] = vidx + (b0 + b) * N
            pltpu.async_copy(x_hbm.at[idxv], rows, sem).wait()
            pltpu.sync_copy(rows, out_hbm.at[pl.ds(b * L, L)])

    return k(score_flat, x_flat)


def _final_body(xs_ref, g_ref, be_ref, wtbf_ref, b_ref, o_ref):
    B = xs_ref.shape[0]
    C = wtbf_ref.shape[1]
    acc = jnp.zeros((B, C), jnp.float32)
    for j in range(L - TOPK, L):
        xr = xs_ref[:, j, :]
        mu = jnp.mean(xr, axis=1, keepdims=True)
        xc = xr - mu
        var = jnp.mean(xc * xc, axis=1, keepdims=True)
        xn = xc / jnp.sqrt(var + EPS) * g_ref[0] + be_ref[0]
        acc = acc + jnp.dot(xn.astype(jnp.bfloat16), wtbf_ref[...],
                            preferred_element_type=jnp.float32)
    o_ref[...] = acc * (1.0 / TOPK) + b_ref[0]


def _final(xsel, g2, be2, wt, b2):
    B = xsel.shape[0]
    D = xsel.shape[2]
    C = wt.shape[1]
    return pl.pallas_call(
        _final_body,
        in_specs=[
            pl.BlockSpec((B, L, D), lambda: (0, 0, 0)),
            pl.BlockSpec((1, D), lambda: (0, 0)),
            pl.BlockSpec((1, D), lambda: (0, 0)),
            pl.BlockSpec((D, C), lambda: (0, 0)),
            pl.BlockSpec((1, C), lambda: (0, 0)),
        ],
        out_specs=pl.BlockSpec((B, C), lambda: (0, 0)),
        out_shape=jax.ShapeDtypeStruct((B, C), jnp.float32),
    )(xsel, g2, be2, wt, b2)


def kernel(x, mask, ln_gamma, ln_beta, W, b):
    B, N, D = x.shape
    C = W.shape[0]
    mask_f = mask.reshape(B * N).astype(jnp.float32)
    g2 = ln_gamma.reshape(1, D)
    be2 = ln_beta.reshape(1, D)
    wbf = W.astype(jnp.bfloat16)
    wtbf = wbf.T
    gt = ln_gamma.reshape(D, 1)
    bet = ln_beta.reshape(D, 1)
    bc = b.reshape(C, 1)
    b2 = b.reshape(1, C)
    sg = _scores(x, mask_f, wbf, gt, bet, bc)
    xf = x.reshape(B * N, D)
    xsel = _sc_select(sg, xf, 0, B, N, D)
    return _final(xsel.reshape(B, L, D), g2, be2, wtbf, b2)


# speedup vs baseline: 1.4378x; 1.1453x over previous
import functools

import jax
import jax.numpy as jnp
from jax import lax
from jax.experimental import pallas as pl
from jax.experimental.pallas import tpu as pltpu
from jax.experimental.pallas import tpu_sc as plsc

EPS = 1e-5
L = 16
NC, NS = 2, 16
NW = NC * NS
TOPK = 8


def _score_body(x_ref, m_ref, wbf_ref, gt_ref, bet_ref, bc_ref, s_ref, *, D):
    xbt = x_ref[0].T
    mu = jnp.sum(xbt, axis=0, keepdims=True) * (1.0 / D)
    xc = xbt - mu
    var = jnp.sum(xc * xc, axis=0, keepdims=True) * (1.0 / D)
    xnt = xc / jnp.sqrt(var + EPS) * gt_ref[...] + bet_ref[...]
    xnb = xnt.astype(jnp.bfloat16)
    lgt = jnp.dot(wbf_ref[...], xnb,
                  preferred_element_type=jnp.float32)
    lgt = lgt + bc_ref[...]
    sc = jnp.max(lgt, axis=0)
    sc = jnp.where(m_ref[...] != 0, sc, -jnp.inf)
    s_ref[...] = sc


def _scores(x, mask_f, wbf, gt, bet, bc, bpb=4):
    B, N, D = x.shape
    C = wbf.shape[0]
    nblk = B // bpb
    rows = bpb * N
    xr = x.reshape(nblk, rows, D)
    return pl.pallas_call(
        functools.partial(_score_body, D=D),
        grid=(nblk,),
        in_specs=[
            pl.BlockSpec((1, rows, D), lambda i: (i, 0, 0)),
            pl.BlockSpec((rows,), lambda i: (i,)),
            pl.BlockSpec((C, D), lambda i: (0, 0)),
            pl.BlockSpec((D, 1), lambda i: (0, 0)),
            pl.BlockSpec((D, 1), lambda i: (0, 0)),
            pl.BlockSpec((C, 1), lambda i: (0, 0)),
        ],
        out_specs=pl.BlockSpec((rows,), lambda i: (i,)),
        out_shape=jax.ShapeDtypeStruct((B * N,), jnp.float32),
    )(xr, mask_f, wbf, gt, bet, bc)


def _sc_select(score_flat, x_flat, b0, bg, N, D):
    bags_per = bg // NW
    n_chunks = N // L
    mesh = plsc.VectorSubcoreMesh(core_axis_name="c", subcore_axis_name="s")

    @functools.partial(
        pl.kernel,
        mesh=mesh,
        out_type=jax.ShapeDtypeStruct((bg * L, D), jnp.float32),
        scratch_types=[
            pltpu.VMEM((N,), jnp.float32),
            pltpu.VMEM((L,), jnp.int32),
            pltpu.VMEM((L, D), jnp.float32),
            pltpu.SemaphoreType.DMA,
        ],
        compiler_params=pltpu.CompilerParams(needs_layout_passes=False),
    )
    def k(score_hbm, x_hbm, out_hbm, sv, idxv, rows, sem):
        wid = lax.axis_index("s") * NC + lax.axis_index("c")
        for j in range(bags_per):
            b = wid * bags_per + j
            pltpu.sync_copy(score_hbm.at[pl.ds(b * N, N)], sv)

            neg = jnp.full((L,), -jnp.inf, jnp.float32)
            zero = jnp.zeros((L,), jnp.int32)
            init = (neg,) * TOPK + (zero,) * TOPK

            def body(i, carry):
                ts, cs = carry[:TOPK], carry[TOPK:]
                cur = sv[pl.ds(i * L, L)]
                curi = jnp.full((L,), i, jnp.int32)
                nts, ncs = [], []
                for t, c in zip(ts, cs):
                    m = cur > t
                    hi = jnp.maximum(t, cur)
                    lo = jnp.minimum(t, cur)
                    nts.append(hi)
                    ncs.append(jnp.where(m, curi, c))
                    curi = jnp.where(m, c, curi)
                    cur = lo
                return tuple(nts) + tuple(ncs)

            carry = lax.fori_loop(0, n_chunks, body, init)
            ts, cs = carry[:TOPK], carry[TOPK:]

            lane = lax.iota(jnp.int32, L)
            pairs = [plsc.sort_key_val(ts[q], cs[q] * L + lane)
                     for q in range(TOPK)]

            def merge(pa, pb):
                ka, va = pa
                kb, vb = pb
                rk = lax.rev(kb, (0,))
                rv = lax.rev(vb, (0,))
                m = ka >= rk
                hk = jnp.maximum(ka, rk)
                hv = jnp.where(m, va, rv)
                return plsc.sort_key_val(hk, hv)

            while len(pairs) > 1:
                pairs = [merge(pairs[q], pairs[q + 1])
                         for q in range(0, len(pairs), 2)]
            _, vidx = pairs[0]

            idxv[...] = vidx + (b0 + b) * N
            pltpu.async_copy(x_hbm.at[idxv], rows, sem).wait()
            pltpu.sync_copy(rows, out_hbm.at[pl.ds(b * L, L)])

    return k(score_flat, x_flat)


def _final_body(xs_ref, g_ref, be_ref, wtbf_ref, b_ref, o_ref):
    B = xs_ref.shape[0]
    C = wtbf_ref.shape[1]
    acc = jnp.zeros((B, C), jnp.float32)
    for j in range(L - TOPK, L):
        xr = xs_ref[:, j, :]
        mu = jnp.mean(xr, axis=1, keepdims=True)
        xc = xr - mu
        var = jnp.mean(xc * xc, axis=1, keepdims=True)
        xn = xc / jnp.sqrt(var + EPS) * g_ref[0] + be_ref[0]
        acc = acc + jnp.dot(xn.astype(jnp.bfloat16), wtbf_ref[...],
                            preferred_element_type=jnp.float32)
    o_ref[...] = acc * (1.0 / TOPK) + b_ref[0]


def _final(xsel, g2, be2, wt, b2):
    B = xsel.shape[0]
    D = xsel.shape[2]
    C = wt.shape[1]
    return pl.pallas_call(
        _final_body,
        in_specs=[
            pl.BlockSpec((B, L, D), lambda: (0, 0, 0)),
            pl.BlockSpec((1, D), lambda: (0, 0)),
            pl.BlockSpec((1, D), lambda: (0, 0)),
            pl.BlockSpec((D, C), lambda: (0, 0)),
            pl.BlockSpec((1, C), lambda: (0, 0)),
        ],
        out_specs=pl.BlockSpec((B, C), lambda: (0, 0)),
        out_shape=jax.ShapeDtypeStruct((B, C), jnp.float32),
    )(xsel, g2, be2, wt, b2)


def kernel(x, mask, ln_gamma, ln_beta, W, b):
    B, N, D = x.shape
    C = W.shape[0]
    mask_f = mask.reshape(B * N).astype(jnp.float32)
    g2 = ln_gamma.reshape(1, D)
    be2 = ln_beta.reshape(1, D)
    wbf = W.astype(jnp.bfloat16)
    wtbf = wbf.T
    gt = ln_gamma.reshape(D, 1)
    bet = ln_beta.reshape(D, 1)
    bc = b.reshape(C, 1)
    b2 = b.reshape(1, C)
    sg = _scores(x, mask_f, wbf, gt, bet, bc)
    xf = x.reshape(B * N, D)
    xsel = _sc_select(sg, xf, 0, B, N, D)
    return _final(xsel.reshape(B, L, D), g2, be2, wtbf, b2)
